# baseline (device time: 1004825 ns/iter reference)
import functools

import jax
import jax.numpy as jnp
from jax import lax
from jax.experimental import pallas as pl
from jax.experimental.pallas import tpu as pltpu

K = 32
KC = 16


def kernel(x):
    m, n = x.shape
    h = m // 2
    rc = h // K
    lc = m // KC

    def body(
        x_ref,
        out_ref,
        ysend,
        yrecv,
        xsend,
        xrecv,
        vbuf,
        isems,
        osems,
    ):
        my_x = lax.axis_index("x")
        my_y = lax.axis_index("y")
        my_z = lax.axis_index("z")
        peer_y = (my_x, 1 - my_y, my_z)
        peer_x = (1 - my_x, my_y, my_z)

        barrier = pltpu.get_barrier_semaphore()
        for nbr in (peer_y, peer_x):
            pl.semaphore_signal(
                barrier, inc=1, device_id=nbr, device_id_type=pl.DeviceIdType.MESH
            )
        pl.semaphore_wait(barrier, 2)

        mine_out = my_y * m
        frn_out = (1 - my_y) * m
        half = my_x * h

        y_rdmas = []
        for k in range(K):
            off = half + k * rc
            r = pltpu.make_async_remote_copy(
                src_ref=x_ref.at[pl.ds(off, rc), :],
                dst_ref=out_ref.at[pl.ds(mine_out + off, rc), :],
                send_sem=ysend.at[k],
                recv_sem=yrecv.at[k],
                device_id=peer_y,
                device_id_type=pl.DeviceIdType.MESH,
            )
            r.start()
            y_rdmas.append(r)

        in_cps = [
            pltpu.make_async_copy(
                x_ref.at[pl.ds(k * lc, lc), :], vbuf.at[k % 2], isems.at[k % 2]
            )
            for k in range(KC)
        ]
        out_cps = [
            pltpu.make_async_copy(
                vbuf.at[k % 2],
                out_ref.at[pl.ds(mine_out + k * lc, lc), :],
                osems.at[k % 2],
            )
            for k in range(KC)
        ]

        def stage_step(k):
            if k >= KC:
                return
            if k == 0:
                in_cps[0].start()
            in_cps[k].wait()
            out_cps[k].start()
            if k + 1 < KC:
                if k >= 1:
                    out_cps[k - 1].wait()
                in_cps[k + 1].start()

        x_rdmas = []
        for k in range(K):
            y_rdmas[k].wait_recv()
            off = frn_out + half + k * rc
            r = pltpu.make_async_remote_copy(
                src_ref=out_ref.at[pl.ds(off, rc), :],
                dst_ref=out_ref.at[pl.ds(off, rc), :],
                send_sem=xsend.at[k],
                recv_sem=xrecv.at[k],
                device_id=peer_x,
                device_id_type=pl.DeviceIdType.MESH,
            )
            r.start()
            x_rdmas.append(r)

        for k in range(KC):
            stage_step(k)
        for k in range(K):
            y_rdmas[k].wait_send()
            x_rdmas[k].wait_send()
            x_rdmas[k].wait_recv()
        if KC >= 2:
            out_cps[KC - 2].wait()
        out_cps[KC - 1].wait()

        @functools.partial(
            pl.run_scoped, second_barrier=pltpu.SemaphoreType.REGULAR
        )
        def _(second_barrier):
            for nbr in (peer_y, peer_x):
                pl.semaphore_signal(
                    second_barrier,
                    inc=1,
                    device_id=nbr,
                    device_id_type=pl.DeviceIdType.MESH,
                )
            pl.semaphore_wait(second_barrier, 2)

    return pl.pallas_call(
        body,
        out_shape=jax.ShapeDtypeStruct((2 * m, n), x.dtype),
        in_specs=[pl.BlockSpec(memory_space=pl.ANY)],
        out_specs=pl.BlockSpec(memory_space=pl.ANY),
        scratch_shapes=[
            pltpu.SemaphoreType.DMA((K,)),
            pltpu.SemaphoreType.DMA((K,)),
            pltpu.SemaphoreType.DMA((K,)),
            pltpu.SemaphoreType.DMA((K,)),
            pltpu.VMEM((2, m // KC, n), jnp.float32),
            pltpu.SemaphoreType.DMA((2,)),
            pltpu.SemaphoreType.DMA((2,)),
        ],
        compiler_params=pltpu.CompilerParams(collective_id=0),
    )(x)


# device time: 931706 ns/iter; 1.0785x vs baseline; 1.0785x over previous
import functools

import jax
import jax.numpy as jnp
from jax import lax
from jax.experimental import pallas as pl
from jax.experimental.pallas import tpu as pltpu

K = 32
GR = 1024


def kernel(x):
    m, n = x.shape
    h = m // 2
    rc = h // K
    cpg = GR // rc
    gh = h // GR
    go = m // GR

    def body(
        x_ref,
        out_ref,
        comm,
        ysend,
        yrecv,
        xsend,
        xrecv,
        vown,
        vy,
        vx,
        oi,
        oo,
        yi,
        yo,
        xi,
        xo,
    ):
        my_x = lax.axis_index("x")
        my_y = lax.axis_index("y")
        my_z = lax.axis_index("z")
        peer_y = (my_x, 1 - my_y, my_z)
        peer_x = (1 - my_x, my_y, my_z)

        barrier = pltpu.get_barrier_semaphore()
        for nbr in (peer_y, peer_x):
            pl.semaphore_signal(
                barrier, inc=1, device_id=nbr, device_id_type=pl.DeviceIdType.MESH
            )
        pl.semaphore_wait(barrier, 2)

        mine = my_y * m
        frn = (1 - my_y) * m
        yhb = my_x * h
        xhb = (1 - my_x) * h

        y_rdmas = []
        for k in range(K):
            r = pltpu.make_async_remote_copy(
                src_ref=x_ref.at[pl.ds(yhb + k * rc, rc), :],
                dst_ref=comm.at[pl.ds(yhb + k * rc, rc), :],
                send_sem=ysend.at[k],
                recv_sem=yrecv.at[k],
                device_id=peer_y,
                device_id_type=pl.DeviceIdType.MESH,
            )
            r.start()
            y_rdmas.append(r)

        own_in = [
            pltpu.make_async_copy(
                x_ref.at[pl.ds(g * GR, GR), :], vown.at[g % 2], oi.at[g % 2]
            )
            for g in range(go)
        ]
        own_out = [
            pltpu.make_async_copy(
                vown.at[g % 2],
                out_ref.at[pl.ds(mine + g * GR, GR), :],
                oo.at[g % 2],
            )
            for g in range(go)
        ]

        def own_step(g):
            if g == 0:
                own_in[0].start()
            own_in[g].wait()
            own_out[g].start()
            if g + 1 < go:
                if g >= 1:
                    own_out[g - 1].wait()
                own_in[g + 1].start()

        fy_in = [
            pltpu.make_async_copy(
                comm.at[pl.ds(yhb + g * GR, GR), :], vy.at[g % 2], yi.at[g % 2]
            )
            for g in range(gh)
        ]
        fy_out = [
            pltpu.make_async_copy(
                vy.at[g % 2],
                out_ref.at[pl.ds(frn + yhb + g * GR, GR), :],
                yo.at[g % 2],
            )
            for g in range(gh)
        ]
        fx_in = [
            pltpu.make_async_copy(
                comm.at[pl.ds(xhb + g * GR, GR), :], vx.at[g % 2], xi.at[g % 2]
            )
            for g in range(gh)
        ]
        fx_out = [
            pltpu.make_async_copy(
                vx.at[g % 2],
                out_ref.at[pl.ds(frn + xhb + g * GR, GR), :],
                xo.at[g % 2],
            )
            for g in range(gh)
        ]

        x_rdmas = []
        x_recv_waited = 0
        fy_in_n = fy_out_n = fx_in_n = fx_out_n = 0
        for k in range(K):
            y_rdmas[k].wait_recv()
            r = pltpu.make_async_remote_copy(
                src_ref=comm.at[pl.ds(yhb + k * rc, rc), :],
                dst_ref=comm.at[pl.ds(yhb + k * rc, rc), :],
                send_sem=xsend.at[k],
                recv_sem=xrecv.at[k],
                device_id=peer_x,
                device_id_type=pl.DeviceIdType.MESH,
            )
            r.start()
            x_rdmas.append(r)

            own_step(k)
            while fy_in_n < min(gh, (k + 1) // cpg):
                g = fy_in_n
                if g >= 2:
                    fy_out[g - 2].wait()
                fy_in[g].start()
                fy_in_n += 1
            while fy_out_n < fy_in_n - 1:
                g = fy_out_n
                fy_in[g].wait()
                fy_out[g].start()
                fy_out_n += 1
            while x_recv_waited < min(len(x_rdmas), max(0, k - 1)):
                x_rdmas[x_recv_waited].wait_recv()
                x_recv_waited += 1
            while fx_in_n < min(gh, x_recv_waited // cpg):
                g = fx_in_n
                if g >= 2:
                    fx_out[g - 2].wait()
                fx_in[g].start()
                fx_in_n += 1
            while fx_out_n < fx_in_n - 1:
                g = fx_out_n
                fx_in[g].wait()
                fx_out[g].start()
                fx_out_n += 1

        while fy_out_n < gh:
            g = fy_out_n
            if fy_in_n <= g:
                if g >= 2:
                    fy_out[g - 2].wait()
                fy_in[g].start()
                fy_in_n += 1
            fy_in[g].wait()
            fy_out[g].start()
            fy_out_n += 1
        while x_recv_waited < K:
            x_rdmas[x_recv_waited].wait_recv()
            x_recv_waited += 1
        while fx_out_n < gh:
            g = fx_out_n
            if fx_in_n <= g:
                if g >= 2:
                    fx_out[g - 2].wait()
                fx_in[g].start()
                fx_in_n += 1
            fx_in[g].wait()
            fx_out[g].start()
            fx_out_n += 1

        for k in range(K):
            y_rdmas[k].wait_send()
            x_rdmas[k].wait_send()
        own_out[go - 2].wait()
        own_out[go - 1].wait()
        fy_out[gh - 2].wait()
        fy_out[gh - 1].wait()
        fx_out[gh - 2].wait()
        fx_out[gh - 1].wait()

        @functools.partial(
            pl.run_scoped, second_barrier=pltpu.SemaphoreType.REGULAR
        )
        def _(second_barrier):
            for nbr in (peer_y, peer_x):
                pl.semaphore_signal(
                    second_barrier,
                    inc=1,
                    device_id=nbr,
                    device_id_type=pl.DeviceIdType.MESH,
                )
            pl.semaphore_wait(second_barrier, 2)

    out, _ = pl.pallas_call(
        body,
        out_shape=(
            jax.ShapeDtypeStruct((2 * m, n), x.dtype),
            jax.ShapeDtypeStruct((m, n), x.dtype),
        ),
        in_specs=[pl.BlockSpec(memory_space=pl.ANY)],
        out_specs=(
            pl.BlockSpec(memory_space=pl.ANY),
            pl.BlockSpec(memory_space=pl.ANY),
        ),
        scratch_shapes=[
            pltpu.SemaphoreType.DMA((K,)),
            pltpu.SemaphoreType.DMA((K,)),
            pltpu.SemaphoreType.DMA((K,)),
            pltpu.SemaphoreType.DMA((K,)),
            pltpu.VMEM((2, GR, n), jnp.float32),
            pltpu.VMEM((2, GR, n), jnp.float32),
            pltpu.VMEM((2, GR, n), jnp.float32),
            pltpu.SemaphoreType.DMA((2,)),
            pltpu.SemaphoreType.DMA((2,)),
            pltpu.SemaphoreType.DMA((2,)),
            pltpu.SemaphoreType.DMA((2,)),
            pltpu.SemaphoreType.DMA((2,)),
            pltpu.SemaphoreType.DMA((2,)),
        ],
        compiler_params=pltpu.CompilerParams(collective_id=0),
    )(x)
    return out


# device time: 926115 ns/iter; 1.0850x vs baseline; 1.0060x over previous
import functools

import jax
import jax.numpy as jnp
from jax import lax
from jax.experimental import pallas as pl
from jax.experimental.pallas import tpu as pltpu

K = 32
KC = 16


def kernel(x):
    m, n = x.shape
    h = m // 2
    rc = h // K
    lc = m // KC

    def body(
        x_ref,
        out_ref,
        ysend,
        yrecv,
        xsend,
        xrecv,
        vbuf,
        isems,
        osems,
    ):
        my_x = lax.axis_index("x")
        my_y = lax.axis_index("y")
        my_z = lax.axis_index("z")
        peer_y = (my_x, 1 - my_y, my_z)
        peer_x = (1 - my_x, my_y, my_z)

        barrier = pltpu.get_barrier_semaphore()
        for nbr in (peer_y, peer_x):
            pl.semaphore_signal(
                barrier, inc=1, device_id=nbr, device_id_type=pl.DeviceIdType.MESH
            )
        pl.semaphore_wait(barrier, 2)

        mine_out = my_y * m
        frn_out = (1 - my_y) * m
        half = my_x * h

        y_rdmas = []
        for k in range(K):
            off = half + k * rc
            r = pltpu.make_async_remote_copy(
                src_ref=x_ref.at[pl.ds(off, rc), :],
                dst_ref=out_ref.at[pl.ds(mine_out + off, rc), :],
                send_sem=ysend.at[k],
                recv_sem=yrecv.at[k],
                device_id=peer_y,
                device_id_type=pl.DeviceIdType.MESH,
            )
            r.start()
            y_rdmas.append(r)

        in_cps = [
            pltpu.make_async_copy(
                x_ref.at[pl.ds(k * lc, lc), :], vbuf.at[k % 2], isems.at[k % 2]
            )
            for k in range(KC)
        ]
        out_cps = [
            pltpu.make_async_copy(
                vbuf.at[k % 2],
                out_ref.at[pl.ds(mine_out + k * lc, lc), :],
                osems.at[k % 2],
            )
            for k in range(KC)
        ]

        def stage_step(k):
            if k >= KC:
                return
            if k == 0:
                in_cps[0].start()
            in_cps[k].wait()
            out_cps[k].start()
            if k + 1 < KC:
                if k >= 1:
                    out_cps[k - 1].wait()
                in_cps[k + 1].start()

        x_rdmas = []
        for k in range(K):
            y_rdmas[k].wait_recv()
            off = frn_out + half + k * rc
            r = pltpu.make_async_remote_copy(
                src_ref=out_ref.at[pl.ds(off, rc), :],
                dst_ref=out_ref.at[pl.ds(off, rc), :],
                send_sem=xsend.at[k],
                recv_sem=xrecv.at[k],
                device_id=peer_x,
                device_id_type=pl.DeviceIdType.MESH,
            )
            r.start()
            x_rdmas.append(r)
            stage_step(k)

        for k in range(K, KC):
            stage_step(k)
        for k in range(K):
            y_rdmas[k].wait_send()
            x_rdmas[k].wait_send()
            x_rdmas[k].wait_recv()
        if KC >= 2:
            out_cps[KC - 2].wait()
        out_cps[KC - 1].wait()

        @functools.partial(
            pl.run_scoped, second_barrier=pltpu.SemaphoreType.REGULAR
        )
        def _(second_barrier):
            for nbr in (peer_y, peer_x):
                pl.semaphore_signal(
                    second_barrier,
                    inc=1,
                    device_id=nbr,
                    device_id_type=pl.DeviceIdType.MESH,
                )
            pl.semaphore_wait(second_barrier, 2)

    return pl.pallas_call(
        body,
        out_shape=jax.ShapeDtypeStruct((2 * m, n), x.dtype),
        in_specs=[pl.BlockSpec(memory_space=pl.ANY)],
        out_specs=pl.BlockSpec(memory_space=pl.ANY),
        scratch_shapes=[
            pltpu.SemaphoreType.DMA((K,)),
            pltpu.SemaphoreType.DMA((K,)),
            pltpu.SemaphoreType.DMA((K,)),
            pltpu.SemaphoreType.DMA((K,)),
            pltpu.VMEM((2, m // KC, n), jnp.float32),
            pltpu.SemaphoreType.DMA((2,)),
            pltpu.SemaphoreType.DMA((2,)),
        ],
        compiler_params=pltpu.CompilerParams(collective_id=0),
    )(x)
